# Initial kernel scaffold; baseline (speedup 1.0000x reference)
import jax
import jax.numpy as jnp
from jax.experimental import pallas as pl


def kernel(x, scene, We1, be1, We2, be2, We3, be3, S, scene_table):
    raise NotImplementedError("write your pallas kernel here")


# speedup vs baseline: 2.1415x; 2.1415x over previous
import jax
import jax.numpy as jnp
from jax.experimental import pallas as pl
from jax.experimental.pallas import tpu as pltpu

_B = 4096
_D = 1024
_E = 7
_H1 = 1024
_H2 = 1024
_HO = 512
_SD = 16
_TB = 1024
_NB = _B // _TB
_G = 49

_INTERPRET = False


def _moe_body(x_ref, se_ref, oh_ref, s2x_ref, s2e_ref,
              w1_ref, b1_ref, w2_ref, b2_ref, w3_ref, b3_ref,
              out_ref, gate_ref):
    e = pl.program_id(1)

    @pl.when(e == 0)
    def _gating():
        hp = jax.lax.Precision.HIGHEST
        xv = x_ref[...]
        raw = (jnp.dot(xv, s2x_ref[...], preferred_element_type=jnp.float32)
               + jnp.dot(se_ref[...], s2e_ref[...],
                         preferred_element_type=jnp.float32))
        r0 = jax.lax.broadcasted_iota(jnp.int32, (_G, _E), 0)
        r1 = jax.lax.broadcasted_iota(jnp.int32, (_G, _E), 1)
        Rs = (r0 // 7 == r1).astype(jnp.float32)
        c0 = jax.lax.broadcasted_iota(jnp.int32, (_E, _G), 0)
        c1 = jax.lax.broadcasted_iota(jnp.int32, (_E, _G), 1)
        RsT = (c1 // 7 == c0).astype(jnp.float32)
        Tt = (c1 % 7 == c0).astype(jnp.float32)

        ex = jnp.exp(raw)
        Z = jnp.dot(ex, Rs, preferred_element_type=jnp.float32, precision=hp)
        Zb = jnp.dot(Z, RsT, preferred_element_type=jnp.float32, precision=hp)
        G49 = ex / Zb
        LG = jnp.log(G49)

        ohe = oh_ref[...][:, :_E]
        OHb = jnp.dot(ohe, Tt, preferred_element_type=jnp.float32, precision=hp)
        gval = jnp.dot(G49 * OHb, Rs, preferred_element_type=jnp.float32, precision=hp)
        ps = jnp.dot(LG * OHb, Rs, preferred_element_type=jnp.float32, precision=hp)
        qv = jnp.dot(LG, Rs, preferred_element_type=jnp.float32, precision=hp)

        il = jax.lax.broadcasted_iota(jnp.int32, (_TB, _E), 1)
        minp = jnp.min(ps, axis=1, keepdims=True)
        a1 = jnp.max(jnp.where(ps == minp, il, -1), axis=1, keepdims=True)
        minq = jnp.min(qv, axis=1, keepdims=True)
        a2 = jnp.max(jnp.where(qv == minq, il, -1), axis=1, keepdims=True)
        excl = jnp.logical_and(a1 == a2, il == a1)

        gm = jnp.max(gval, axis=1, keepdims=True)
        ge = jnp.exp(gval - gm)
        gate = ge / jnp.sum(ge, axis=1, keepdims=True)
        gate = jnp.where(excl, 0.0, gate)
        gate_ref[...] = jnp.concatenate(
            [gate, jnp.zeros((_TB, 1), jnp.float32)], axis=1)

    h = jnp.maximum(
        jnp.dot(x_ref[...], w1_ref[0], preferred_element_type=jnp.float32)
        + b1_ref[0], 0.0)
    h = jnp.maximum(
        jnp.dot(h, w2_ref[0], preferred_element_type=jnp.float32)
        + b2_ref[0], 0.0)
    o = (jnp.dot(h, w3_ref[0], preferred_element_type=jnp.float32)
         + b3_ref[0])

    emask = (jax.lax.broadcasted_iota(jnp.int32, (_TB, 8), 1) == e)
    g = jnp.sum(gate_ref[...] * emask.astype(jnp.float32),
                axis=1, keepdims=True)

    @pl.when(e == 0)
    def _init():
        out_ref[...] = g * o

    @pl.when(e != 0)
    def _acc():
        out_ref[...] += g * o


def _moe_call(x, se, oh, s2x, s2e, We1, be1, We2, be2, We3, be3):
    grid = (_NB, _E)
    return pl.pallas_call(
        _moe_body,
        grid=grid,
        in_specs=[
            pl.BlockSpec((_TB, _D), lambda i, e: (i, 0)),
            pl.BlockSpec((_TB, _SD), lambda i, e: (i, 0)),
            pl.BlockSpec((_TB, 8), lambda i, e: (i, 0)),
            pl.BlockSpec((_D, _G), lambda i, e: (0, 0)),
            pl.BlockSpec((_SD, _G), lambda i, e: (0, 0)),
            pl.BlockSpec((1, _D, _H1), lambda i, e: (e, 0, 0)),
            pl.BlockSpec((1, 1, _H1), lambda i, e: (e, 0, 0)),
            pl.BlockSpec((1, _H1, _H2), lambda i, e: (e, 0, 0)),
            pl.BlockSpec((1, 1, _H2), lambda i, e: (e, 0, 0)),
            pl.BlockSpec((1, _H2, _HO), lambda i, e: (e, 0, 0)),
            pl.BlockSpec((1, 1, _HO), lambda i, e: (e, 0, 0)),
        ],
        out_specs=pl.BlockSpec((_TB, _HO), lambda i, e: (i, 0)),
        out_shape=jax.ShapeDtypeStruct((_B, _HO), jnp.float32),
        scratch_shapes=[pltpu.VMEM((_TB, 8), jnp.float32)],
        compiler_params=pltpu.CompilerParams(
            dimension_semantics=("parallel", "arbitrary")),
        interpret=_INTERPRET,
    )(x, se, oh, s2x, s2e, We1, be1, We2, be2, We3, be3)


def kernel(x, scene, We1, be1, We2, be2, We3, be3, S, scene_table):
    scene = scene.astype(jnp.int32)
    se = jnp.take(scene_table, scene, axis=0)
    oh = (scene[:, None] == jnp.arange(8, dtype=jnp.int32)
          ).astype(jnp.float32)

    s2 = S.transpose(1, 2, 0).reshape(_D + _SD, _G)
    s2x, s2e = s2[:_D], s2[_D:]
    b1r = be1.reshape(_E, 1, _H1)
    b2r = be2.reshape(_E, 1, _H2)
    b3r = be3.reshape(_E, 1, _HO)

    y = _moe_call(x, se, oh, s2x, s2e, We1, b1r, We2, b2r, We3, b3r)
    return (y, y, y)
